# baseline (speedup 1.0000x reference)
import functools

import jax
import jax.numpy as jnp
from jax.experimental import pallas as pl
from jax.experimental.pallas import tpu as pltpu

_NJB = 128


def _router_body(x_ref, wg_ref, xb_ref, wft_ref, rkt_ref, be_ref, bb_ref,
                 bv_ref, *, bsz):
    x = x_ref[...]
    t = x.shape[0]
    xb_ref[...] = x.astype(jnp.bfloat16)
    logits = jax.lax.dot_general(
        x, wg_ref[...], (((1,), (1,)), ((), ())),
        preferred_element_type=jnp.float32)
    mx = jnp.max(logits, axis=-1, keepdims=True)
    ex = jnp.exp(logits - mx)
    p = ex / jnp.sum(ex, axis=-1, keepdims=True)
    num_e = p.shape[-1]
    col = jax.lax.broadcasted_iota(jnp.int32, p.shape, 1)
    m1 = jnp.max(p, axis=-1, keepdims=True)
    i1 = jnp.min(jnp.where(p == m1, col, num_e), axis=-1, keepdims=True)
    mask1 = col == i1
    p2 = jnp.where(mask1, -jnp.inf, p)
    m2 = jnp.max(p2, axis=-1, keepdims=True)
    i2 = jnp.min(jnp.where(p2 == m2, col, num_e), axis=-1, keepdims=True)
    mask2 = col == i2
    denom = m1 + m2
    wfull = (jnp.where(mask1, m1, 0.0) + jnp.where(mask2, m2, 0.0)) / denom
    m = jnp.where(mask1 | mask2, 1.0, 0.0)
    ir = jax.lax.broadcasted_iota(jnp.int32, (t, t), 0)
    ic = jax.lax.broadcasted_iota(jnp.int32, (t, t), 1)
    tri = (ir >= ic).astype(jnp.bfloat16)
    rank = jax.lax.dot_general(
        tri, m.astype(jnp.bfloat16), (((1,), (0,)), ((), ())),
        preferred_element_type=jnp.float32)
    wft_ref[...] = wfull.T
    rkt_ref[...] = (rank * m).T.astype(jnp.int32)
    counts = jnp.sum(m, axis=0, keepdims=True)
    nb = jnp.floor((counts + (bsz - 1)) / bsz)
    nb8 = jnp.broadcast_to(nb, (num_e, num_e))
    sr = jax.lax.broadcasted_iota(jnp.int32, (num_e, num_e), 0)
    sc = jax.lax.broadcasted_iota(jnp.int32, (num_e, num_e), 1)
    bbex = jnp.sum(jnp.where(sc < sr, nb8, 0.0), axis=1,
                   keepdims=True)
    total = jnp.sum(nb)
    jrow = jax.lax.broadcasted_iota(jnp.int32, (num_e, _NJB), 1).astype(jnp.float32)
    bbex_b = jnp.broadcast_to(bbex, (num_e, _NJB))
    ge = (jrow >= bbex_b).astype(jnp.float32)
    e_j = jnp.sum(ge, axis=0, keepdims=True) - 1.0
    sr2 = jax.lax.broadcasted_iota(jnp.int32, (num_e, _NJB), 0).astype(jnp.float32)
    oh = sr2 == jnp.broadcast_to(e_j, (num_e, _NJB))
    bbsel = jnp.sum(jnp.where(oh, bbex_b, 0.0), axis=0, keepdims=True)
    j1 = jax.lax.broadcasted_iota(jnp.int32, (1, _NJB), 1).astype(jnp.float32)
    b_j = (j1 - bbsel) * bsz
    be_ref[...] = e_j.astype(jnp.int32)
    bb_ref[...] = b_j.astype(jnp.int32)
    bv_ref[...] = (j1 < total).astype(jnp.int32)


def _ffn_body(se_ref, sb_ref, sv_ref, xb_ref, rkt_ref, wft_ref, wg_ref,
              wu_ref, wd_ref, out_ref, *, bsz):
    j = pl.program_id(0)
    ej = se_ref[j]
    bj = sb_ref[j]
    vj = sv_ref[j]
    t = xb_ref.shape[0]
    num_e = rkt_ref.shape[0]

    @pl.when(j == 0)
    def _init():
        out_ref[...] = jnp.zeros_like(out_ref)

    @pl.when(vj == 1)
    def _compute():
        srow = jax.lax.broadcasted_iota(jnp.int32, (num_e, t), 0)
        esel = srow == ej
        rcol = jnp.sum(jnp.where(esel, rkt_ref[...], 0), axis=0,
                       keepdims=True)
        wcol = jnp.sum(jnp.where(esel, wft_ref[...], 0.0), axis=0,
                       keepdims=True)
        islot = jax.lax.broadcasted_iota(jnp.int32, (bsz, t), 0)
        sel = jnp.broadcast_to(rcol, (bsz, t)) == (islot + bj + 1)
        pb = sel.astype(jnp.bfloat16)
        xg = jnp.dot(pb, xb_ref[...],
                     preferred_element_type=jnp.float32).astype(jnp.bfloat16)
        g = jnp.dot(xg, wg_ref[0].astype(jnp.bfloat16),
                    preferred_element_type=jnp.float32)
        u = jnp.dot(xg, wu_ref[0].astype(jnp.bfloat16),
                    preferred_element_type=jnp.float32)
        h = (g * jax.nn.sigmoid(g) * u).astype(jnp.bfloat16)
        o = jnp.dot(h, wd_ref[0].astype(jnp.bfloat16),
                    preferred_element_type=jnp.float32)
        wrow = jnp.sum(jnp.where(sel, jnp.broadcast_to(wcol, (bsz, t)), 0.0),
                       axis=1, keepdims=True)
        ow = (o * wrow).astype(jnp.bfloat16)
        out_ref[...] += jax.lax.dot_general(
            pb, ow, (((0,), (0,)), ((), ())),
            preferred_element_type=jnp.float32)


def kernel(hidden_states, Wg, W_gate, W_up, W_down):
    T, D = hidden_states.shape
    E, _, F = W_gate.shape
    K = 2
    B = min(128, T * K)
    NJ = (T * K + B - 1) // B + E - 1
    router = functools.partial(_router_body, bsz=B)
    xb, wft, rkt, be, bb, bv = pl.pallas_call(
        router,
        out_shape=[
            jax.ShapeDtypeStruct((T, D), jnp.bfloat16),
            jax.ShapeDtypeStruct((E, T), jnp.float32),
            jax.ShapeDtypeStruct((E, T), jnp.int32),
            jax.ShapeDtypeStruct((1, _NJB), jnp.int32),
            jax.ShapeDtypeStruct((1, _NJB), jnp.int32),
            jax.ShapeDtypeStruct((1, _NJB), jnp.int32),
        ],
    )(hidden_states, Wg)
    se = be.reshape((_NJB,))
    sb = bb.reshape((_NJB,))
    sv = bv.reshape((_NJB,))
    ffn = functools.partial(_ffn_body, bsz=B)
    grid_spec = pltpu.PrefetchScalarGridSpec(
        num_scalar_prefetch=3,
        grid=(NJ,),
        in_specs=[
            pl.BlockSpec((T, D), lambda j, se, sb, sv: (0, 0)),
            pl.BlockSpec((E, T), lambda j, se, sb, sv: (0, 0)),
            pl.BlockSpec((E, T), lambda j, se, sb, sv: (0, 0)),
            pl.BlockSpec((1, D, F), lambda j, se, sb, sv: (se[j], 0, 0)),
            pl.BlockSpec((1, D, F), lambda j, se, sb, sv: (se[j], 0, 0)),
            pl.BlockSpec((1, F, D), lambda j, se, sb, sv: (se[j], 0, 0)),
        ],
        out_specs=pl.BlockSpec((T, D), lambda j, se, sb, sv: (0, 0)),
    )
    return pl.pallas_call(
        ffn,
        grid_spec=grid_spec,
        out_shape=jax.ShapeDtypeStruct((T, D), jnp.float32),
        compiler_params=pltpu.CompilerParams(
            dimension_semantics=("arbitrary",)),
    )(se, sb, sv, xb, rkt, wft, W_gate, W_up, W_down)


# speedup vs baseline: 1.2491x; 1.2491x over previous
import functools

import jax
import jax.numpy as jnp
from jax.experimental import pallas as pl
from jax.experimental.pallas import tpu as pltpu

_NJB = 128


def _router_body(x_ref, wg_ref, xb_ref, wft_ref, rkt_ref, be_ref, bb_ref,
                 bv_ref, *, bsz):
    x = x_ref[...]
    t = x.shape[0]
    xb_ref[...] = x.astype(jnp.bfloat16)
    logits = jax.lax.dot_general(
        x, wg_ref[...], (((1,), (1,)), ((), ())),
        preferred_element_type=jnp.float32)
    mx = jnp.max(logits, axis=-1, keepdims=True)
    ex = jnp.exp(logits - mx)
    p = ex / jnp.sum(ex, axis=-1, keepdims=True)
    num_e = p.shape[-1]
    col = jax.lax.broadcasted_iota(jnp.int32, p.shape, 1)
    m1 = jnp.max(p, axis=-1, keepdims=True)
    i1 = jnp.min(jnp.where(p == m1, col, num_e), axis=-1, keepdims=True)
    mask1 = col == i1
    p2 = jnp.where(mask1, -jnp.inf, p)
    m2 = jnp.max(p2, axis=-1, keepdims=True)
    i2 = jnp.min(jnp.where(p2 == m2, col, num_e), axis=-1, keepdims=True)
    mask2 = col == i2
    denom = m1 + m2
    wfull = (jnp.where(mask1, m1, 0.0) + jnp.where(mask2, m2, 0.0)) / denom
    m = jnp.where(mask1 | mask2, 1.0, 0.0)
    ir = jax.lax.broadcasted_iota(jnp.int32, (t, t), 0)
    ic = jax.lax.broadcasted_iota(jnp.int32, (t, t), 1)
    tri = (ir >= ic).astype(jnp.bfloat16)
    rank = jax.lax.dot_general(
        tri, m.astype(jnp.bfloat16), (((1,), (0,)), ((), ())),
        preferred_element_type=jnp.float32)
    wft_ref[...] = wfull.T
    rkt_ref[...] = (rank * m).T.astype(jnp.int32)
    counts = jnp.sum(m, axis=0, keepdims=True)
    nb = jnp.floor((counts + (bsz - 1)) / bsz)
    nb8 = jnp.broadcast_to(nb, (num_e, num_e))
    sr = jax.lax.broadcasted_iota(jnp.int32, (num_e, num_e), 0)
    sc = jax.lax.broadcasted_iota(jnp.int32, (num_e, num_e), 1)
    bbex = jnp.sum(jnp.where(sc < sr, nb8, 0.0), axis=1,
                   keepdims=True)
    total = jnp.sum(nb)
    jrow = jax.lax.broadcasted_iota(jnp.int32, (num_e, _NJB), 1).astype(jnp.float32)
    bbex_b = jnp.broadcast_to(bbex, (num_e, _NJB))
    ge = (jrow >= bbex_b).astype(jnp.float32)
    e_j = jnp.sum(ge, axis=0, keepdims=True) - 1.0
    sr2 = jax.lax.broadcasted_iota(jnp.int32, (num_e, _NJB), 0).astype(jnp.float32)
    oh = sr2 == jnp.broadcast_to(e_j, (num_e, _NJB))
    bbsel = jnp.sum(jnp.where(oh, bbex_b, 0.0), axis=0, keepdims=True)
    j1 = jax.lax.broadcasted_iota(jnp.int32, (1, _NJB), 1).astype(jnp.float32)
    b_j = (j1 - bbsel) * bsz
    be_ref[...] = e_j.astype(jnp.int32)
    bb_ref[...] = b_j.astype(jnp.int32)
    bv_ref[...] = (j1 < total).astype(jnp.int32)


def _ffn_body(se_ref, sb_ref, sv_ref, xb_ref, rkt_ref, wft_ref, wg_ref,
              wu_ref, wd_ref, out_ref, *, bsz):
    j = pl.program_id(0)
    ej = se_ref[j]
    bj = sb_ref[j]
    vj = sv_ref[j]
    t = xb_ref.shape[0]
    num_e = rkt_ref.shape[0]

    @pl.when(j == 0)
    def _init():
        out_ref[...] = jnp.zeros_like(out_ref)

    @pl.when(vj == 1)
    def _compute():
        srow = jax.lax.broadcasted_iota(jnp.int32, (num_e, t), 0)
        esel = srow == ej
        rcol = jnp.sum(jnp.where(esel, rkt_ref[...], 0), axis=0,
                       keepdims=True)
        wcol = jnp.sum(jnp.where(esel, wft_ref[...], 0.0), axis=0,
                       keepdims=True)
        islot = jax.lax.broadcasted_iota(jnp.int32, (bsz, t), 0)
        sel = jnp.broadcast_to(rcol, (bsz, t)) == (islot + bj + 1)
        pb = sel.astype(jnp.bfloat16)
        xg = jnp.dot(pb, xb_ref[...],
                     preferred_element_type=jnp.float32).astype(jnp.bfloat16)
        g = jnp.dot(xg, wg_ref[0].astype(jnp.bfloat16),
                    preferred_element_type=jnp.float32)
        u = jnp.dot(xg, wu_ref[0].astype(jnp.bfloat16),
                    preferred_element_type=jnp.float32)
        h = (g * jax.nn.sigmoid(g) * u).astype(jnp.bfloat16)
        o = jnp.dot(h, wd_ref[0].astype(jnp.bfloat16),
                    preferred_element_type=jnp.float32)
        pw = jnp.where(sel, jnp.broadcast_to(wcol, (bsz, t)),
                       0.0).astype(jnp.bfloat16)
        out_ref[...] += jax.lax.dot_general(
            pw, o.astype(jnp.bfloat16), (((0,), (0,)), ((), ())),
            preferred_element_type=jnp.float32)


def kernel(hidden_states, Wg, W_gate, W_up, W_down):
    T, D = hidden_states.shape
    E, _, F = W_gate.shape
    K = 2
    B = min(256, T * K)
    NJ = (T * K + B - 1) // B + E - 1
    router = functools.partial(_router_body, bsz=B)
    xb, wft, rkt, be, bb, bv = pl.pallas_call(
        router,
        out_shape=[
            jax.ShapeDtypeStruct((T, D), jnp.bfloat16),
            jax.ShapeDtypeStruct((E, T), jnp.float32),
            jax.ShapeDtypeStruct((E, T), jnp.int32),
            jax.ShapeDtypeStruct((1, _NJB), jnp.int32),
            jax.ShapeDtypeStruct((1, _NJB), jnp.int32),
            jax.ShapeDtypeStruct((1, _NJB), jnp.int32),
        ],
    )(hidden_states, Wg)
    se = be.reshape((_NJB,))
    sb = bb.reshape((_NJB,))
    sv = bv.reshape((_NJB,))
    ffn = functools.partial(_ffn_body, bsz=B)
    grid_spec = pltpu.PrefetchScalarGridSpec(
        num_scalar_prefetch=3,
        grid=(NJ,),
        in_specs=[
            pl.BlockSpec((T, D), lambda j, se, sb, sv: (0, 0)),
            pl.BlockSpec((E, T), lambda j, se, sb, sv: (0, 0)),
            pl.BlockSpec((E, T), lambda j, se, sb, sv: (0, 0)),
            pl.BlockSpec((1, D, F), lambda j, se, sb, sv: (se[j], 0, 0)),
            pl.BlockSpec((1, D, F), lambda j, se, sb, sv: (se[j], 0, 0)),
            pl.BlockSpec((1, F, D), lambda j, se, sb, sv: (se[j], 0, 0)),
        ],
        out_specs=pl.BlockSpec((T, D), lambda j, se, sb, sv: (0, 0)),
    )
    return pl.pallas_call(
        ffn,
        grid_spec=grid_spec,
        out_shape=jax.ShapeDtypeStruct((T, D), jnp.float32),
        compiler_params=pltpu.CompilerParams(
            dimension_semantics=("arbitrary",)),
    )(se, sb, sv, xb, rkt, wft, W_gate, W_up, W_down)
